# Initial kernel scaffold; baseline (speedup 1.0000x reference)
import jax
import jax.numpy as jnp
from jax.experimental import pallas as pl


def kernel(x, edge_index, batch, W1, b1, W2, b2, Wc, bc):
    raise NotImplementedError("write your pallas kernel here")


# speedup vs baseline: 19.5477x; 19.5477x over previous
import functools

import jax
import jax.numpy as jnp
from jax import lax
from jax.experimental import pallas as pl
from jax.experimental.pallas import tpu as pltpu
from jax.experimental.pallas import tpu_sc as plsc

_N = 10000
_E = 320000
_D = 128
_G = 64
_C = 32

_NB = 25
_BN = _N // _NB
_NW = 32
_EW = _E // _NW
_KC = 80
_NCH = _EW // _KC
_NP = 10240
_ST = 640

_sc_mesh = plsc.VectorSubcoreMesh(core_axis_name="c", subcore_axis_name="s")


@functools.partial(
    pl.kernel,
    mesh=_sc_mesh,
    out_type=jax.ShapeDtypeStruct((2, _NP), jnp.float32),
    scratch_types=[
        pltpu.VMEM((_NCH, _KC), jnp.int32),
        pltpu.VMEM((_KC,), jnp.float32),
        pltpu.VMEM((_ST,), jnp.float32),
        pltpu.VMEM_SHARED((_NP,), jnp.float32)
    ],
)
def _deg_kernel(dst_hbm, out_hbm, dstv, onesv, zb, hist):
    cid = lax.axis_index("c")
    sid = lax.axis_index("s")
    wid = cid * 16 + sid
    pltpu.sync_copy(dst_hbm.at[wid], dstv)

    def _fill_ones(i, carry):
        onesv[pl.ds(i * 16, 16)] = jnp.full((16,), 1.0, jnp.float32)
        return carry

    lax.fori_loop(0, _KC // 16, _fill_ones, 0)

    def _fill_zero(i, carry):
        zb[pl.ds(i * 16, 16)] = jnp.zeros((16,), jnp.float32)
        return carry

    lax.fori_loop(0, _ST // 16, _fill_zero, 0)

    pltpu.sync_copy(zb, hist.at[pl.ds(sid * _ST, _ST)])
    plsc.subcore_barrier()

    def _step(j, carry):
        pltpu.sync_copy(onesv, hist.at[dstv.at[j]], add=True)
        return carry

    lax.fori_loop(0, _NCH, _step, 0)
    plsc.subcore_barrier()
    pltpu.sync_copy(hist.at[pl.ds(sid * _ST, _ST)],
                    out_hbm.at[cid, pl.ds(sid * _ST, _ST)])


@functools.partial(
    pl.kernel,
    mesh=_sc_mesh,
    out_type=jax.ShapeDtypeStruct((2, _N, _D), jnp.float32),
    scratch_types=[
        pltpu.VMEM((_NCH, _KC), jnp.int32),
        pltpu.VMEM((_NCH, _KC), jnp.int32),
        pltpu.VMEM((_KC, _D), jnp.float32),
        pltpu.VMEM_SHARED((_N, _D), jnp.float32),
        pltpu.SemaphoreType.DMA,
    ],
)
def _agg_kernel(y_hbm, src_hbm, dst_hbm, out_hbm,
                srcv, dstv, rows, acc, sem):
    cid = lax.axis_index("c")
    sid = lax.axis_index("s")
    wid = cid * 16 + sid
    pltpu.sync_copy(src_hbm.at[wid], srcv)
    pltpu.sync_copy(dst_hbm.at[wid], dstv)

    def _z(i, carry):
        for j in range(_D // 16):
            rows[i, pl.ds(j * 16, 16)] = jnp.zeros((16,), jnp.float32)
        return carry

    lax.fori_loop(0, _KC, _z, 0)
    base = sid * _ST

    @pl.when(sid < 15)
    def _():
        for q in range(8):
            pltpu.sync_copy(rows, acc.at[pl.ds(base + q * _KC, _KC)])

    @pl.when(sid == 15)
    def _():
        for q in range(5):
            pltpu.sync_copy(rows, acc.at[pl.ds(9600 + q * _KC, _KC)])

    plsc.subcore_barrier()

    def _step(j, carry):
        pltpu.async_copy(y_hbm.at[srcv.at[j]], rows, sem).wait()
        pltpu.sync_copy(rows, acc.at[dstv.at[j]], add=True)
        return carry

    lax.fori_loop(0, _NCH, _step, 0)
    plsc.subcore_barrier()

    @pl.when(sid < 15)
    def _():
        pltpu.sync_copy(acc.at[pl.ds(base, _ST)],
                        out_hbm.at[cid, pl.ds(base, _ST)])

    @pl.when(sid == 15)
    def _():
        pltpu.sync_copy(acc.at[pl.ds(9600, 400)],
                        out_hbm.at[cid, pl.ds(9600, 400)])


def _m1_body(degp_ref, x_ref, w_ref, y_ref, dinv_ref):
    deg = degp_ref[0, 0, :] + degp_ref[0, 1, :] + 1.0
    dinv = lax.rsqrt(deg)
    xw = jnp.dot(x_ref[...], w_ref[...], preferred_element_type=jnp.float32)
    y_ref[...] = xw * dinv[:, None]
    dinv_ref[0, 0, :] = dinv


_m1 = pl.pallas_call(
    _m1_body,
    grid=(_NB,),
    in_specs=[
        pl.BlockSpec((1, 2, _BN), lambda i: (i, 0, 0)),
        pl.BlockSpec((_BN, _D), lambda i: (i, 0)),
        pl.BlockSpec((_D, _D), lambda i: (0, 0)),
    ],
    out_specs=[
        pl.BlockSpec((_BN, _D), lambda i: (i, 0)),
        pl.BlockSpec((1, 1, _BN), lambda i: (i, 0, 0)),
    ],
    out_shape=[
        jax.ShapeDtypeStruct((_N, _D), jnp.float32),
        jax.ShapeDtypeStruct((_NB, 1, _BN), jnp.float32),
    ],
)


def _m2_body(p_ref, y1_ref, dinv_ref, b1_ref, w_ref, y2_ref):
    dinv = dinv_ref[0, 0, :]
    h = (p_ref[0] + p_ref[1] + y1_ref[...]) * dinv[:, None] + b1_ref[...]
    h = jnp.maximum(h, 0.0)
    xw = jnp.dot(h, w_ref[...], preferred_element_type=jnp.float32)
    y2_ref[...] = xw * dinv[:, None]


_m2 = pl.pallas_call(
    _m2_body,
    grid=(_NB,),
    in_specs=[
        pl.BlockSpec((2, _BN, _D), lambda i: (0, i, 0)),
        pl.BlockSpec((_BN, _D), lambda i: (i, 0)),
        pl.BlockSpec((1, 1, _BN), lambda i: (i, 0, 0)),
        pl.BlockSpec((1, _D), lambda i: (0, 0)),
        pl.BlockSpec((_D, _D), lambda i: (0, 0)),
    ],
    out_specs=pl.BlockSpec((_BN, _D), lambda i: (i, 0)),
    out_shape=jax.ShapeDtypeStruct((_N, _D), jnp.float32),
)


def _m3_body(p_ref, y2_ref, dinv_ref, b2_ref, batch_ref, wc_ref, bc_ref,
             out_ref, psum, cnt):
    i = pl.program_id(0)

    @pl.when(i == 0)
    def _():
        psum[...] = jnp.zeros_like(psum)
        cnt[...] = jnp.zeros_like(cnt)

    dinv = dinv_ref[0, 0, :]
    h = (p_ref[0] + p_ref[1] + y2_ref[...]) * dinv[:, None] + b2_ref[...]
    h = jnp.maximum(h, 0.0)
    bb = batch_ref[0, 0, :]
    sel = (lax.broadcasted_iota(jnp.int32, (_G, _BN), 0) == bb[None, :])
    sel = sel.astype(jnp.float32)
    psum[...] += jnp.dot(sel, h, preferred_element_type=jnp.float32)
    cnt[...] += jnp.broadcast_to(jnp.sum(sel, axis=1, keepdims=True), (_G, _D))

    @pl.when(i == _NB - 1)
    def _():
        pooled = psum[...] / jnp.maximum(cnt[...], 1.0)
        out_ref[...] = lax.dot_general(
            pooled, wc_ref[...], (((1,), (1,)), ((), ())),
            preferred_element_type=jnp.float32) + bc_ref[...]


_m3 = pl.pallas_call(
    _m3_body,
    grid=(_NB,),
    in_specs=[
        pl.BlockSpec((2, _BN, _D), lambda i: (0, i, 0)),
        pl.BlockSpec((_BN, _D), lambda i: (i, 0)),
        pl.BlockSpec((1, 1, _BN), lambda i: (i, 0, 0)),
        pl.BlockSpec((1, _D), lambda i: (0, 0)),
        pl.BlockSpec((1, 1, _BN), lambda i: (i, 0, 0)),
        pl.BlockSpec((_C, _D), lambda i: (0, 0)),
        pl.BlockSpec((1, _C), lambda i: (0, 0)),
    ],
    out_specs=pl.BlockSpec((_G, _C), lambda i: (0, 0)),
    out_shape=jax.ShapeDtypeStruct((_G, _C), jnp.float32),
    scratch_shapes=[
        pltpu.VMEM((_G, _D), jnp.float32),
        pltpu.VMEM((_G, _D), jnp.float32),
    ],
)


def kernel(x, edge_index, batch, W1, b1, W2, b2, Wc, bc):
    src = edge_index[0].reshape(_NW, _NCH, _KC)
    dst = edge_index[1].reshape(_NW, _NCH, _KC)
    degp = _deg_kernel(dst)
    degp_t = degp[:, :_N].reshape(2, _NB, _BN).transpose(1, 0, 2)
    y1, dinv3 = _m1(degp_t, x, W1)
    p1 = _agg_kernel(y1, src, dst)
    y2 = _m2(p1, y1, dinv3, b1.reshape(1, _D), W2)
    p2 = _agg_kernel(y2, src, dst)
    batch3 = batch.reshape(_NB, 1, _BN)
    return _m3(p2, y2, dinv3, b2.reshape(1, _D), batch3, Wc,
               bc.reshape(1, _C))


# speedup vs baseline: 30.0787x; 1.5387x over previous
import functools

import jax
import jax.numpy as jnp
from jax import lax
from jax.experimental import pallas as pl
from jax.experimental.pallas import tpu as pltpu
from jax.experimental.pallas import tpu_sc as plsc

_N = 10000
_E = 320000
_D = 128
_G = 64
_C = 32

_NB = 25
_BN = _N // _NB
_NW = 32
_EW = _E // _NW
_KC = 125
_NCH = _EW // _KC
_NP = 10240
_ST = 640

_sc_mesh = plsc.VectorSubcoreMesh(core_axis_name="c", subcore_axis_name="s")


@functools.partial(
    pl.kernel,
    mesh=_sc_mesh,
    out_type=jax.ShapeDtypeStruct((2, _NP), jnp.float32),
    scratch_types=[
        pltpu.VMEM((_NCH, _KC), jnp.int32),
        pltpu.VMEM((128,), jnp.float32),
        pltpu.VMEM((_ST,), jnp.float32),
        pltpu.VMEM_SHARED((_NP,), jnp.float32)
    ],
)
def _deg_kernel(dst_hbm, out_hbm, dstv, onesv, zb, hist):
    cid = lax.axis_index("c")
    sid = lax.axis_index("s")
    wid = cid * 16 + sid
    pltpu.sync_copy(dst_hbm.at[wid], dstv)

    def _fill_ones(i, carry):
        onesv[pl.ds(i * 16, 16)] = jnp.full((16,), 1.0, jnp.float32)
        return carry

    lax.fori_loop(0, 8, _fill_ones, 0)

    def _fill_zero(i, carry):
        zb[pl.ds(i * 16, 16)] = jnp.zeros((16,), jnp.float32)
        return carry

    lax.fori_loop(0, _ST // 16, _fill_zero, 0)

    pltpu.sync_copy(zb, hist.at[pl.ds(sid * _ST, _ST)])
    plsc.subcore_barrier()

    def _step(j, carry):
        pltpu.sync_copy(onesv.at[pl.ds(0, _KC)], hist.at[dstv.at[j]], add=True)
        return carry

    lax.fori_loop(0, _NCH, _step, 0)
    plsc.subcore_barrier()
    pltpu.sync_copy(hist.at[pl.ds(sid * _ST, _ST)],
                    out_hbm.at[cid, pl.ds(sid * _ST, _ST)])


_NSL = 5
_SCH = _NCH // _NSL


@functools.partial(
    pl.kernel,
    mesh=_sc_mesh,
    out_type=jax.ShapeDtypeStruct((2, _N, _D), jnp.float32),
    scratch_types=[
        pltpu.VMEM((_SCH, _KC), jnp.int32),
        pltpu.VMEM((_SCH, _KC), jnp.int32),
        pltpu.VMEM((_KC, _D), jnp.float32),
        pltpu.VMEM((_KC, _D), jnp.float32),
        pltpu.VMEM_SHARED((_N, _D), jnp.float32),
        pltpu.SemaphoreType.DMA,
        pltpu.SemaphoreType.DMA,
    ],
)
def _agg_kernel(y_hbm, src_hbm, dst_hbm, out_hbm,
                srcv, dstv, rows0, rows1, acc, sem0, sem1):
    cid = lax.axis_index("c")
    sid = lax.axis_index("s")
    wid = cid * 16 + sid

    def _z(i, carry):
        for j in range(_D // 16):
            rows0[i, pl.ds(j * 16, 16)] = jnp.zeros((16,), jnp.float32)
            rows1[i, pl.ds(j * 16, 16)] = jnp.zeros((16,), jnp.float32)
        return carry

    lax.fori_loop(0, _KC, _z, 0)
    base = sid * _ST

    @pl.when(sid < 15)
    def _():
        for q in range(8):
            pltpu.sync_copy(rows0.at[pl.ds(0, 80)],
                            acc.at[pl.ds(base + q * 80, 80)])

    @pl.when(sid == 15)
    def _():
        for q in range(5):
            pltpu.sync_copy(rows0.at[pl.ds(0, 80)],
                            acc.at[pl.ds(9600 + q * 80, 80)])

    plsc.subcore_barrier()

    for s in range(_NSL):
        pltpu.sync_copy(src_hbm.at[wid, pl.ds(s * _SCH, _SCH)], srcv)
        pltpu.sync_copy(dst_hbm.at[wid, pl.ds(s * _SCH, _SCH)], dstv)
        pltpu.async_copy(y_hbm.at[srcv.at[0]], rows0, sem0)

        def _pair(i, carry):
            k = 2 * i
            pltpu.async_copy(y_hbm.at[srcv.at[k + 1]], rows1, sem1)
            pltpu.make_async_copy(y_hbm.at[srcv.at[k]], rows0, sem0).wait()
            pltpu.sync_copy(rows0, acc.at[dstv.at[k]], add=True)
            pltpu.async_copy(y_hbm.at[srcv.at[k + 2]], rows0, sem0)
            pltpu.make_async_copy(y_hbm.at[srcv.at[k + 1]], rows1, sem1).wait()
            pltpu.sync_copy(rows1, acc.at[dstv.at[k + 1]], add=True)
            return carry

        lax.fori_loop(0, (_SCH - 2) // 2, _pair, 0)
        pltpu.async_copy(y_hbm.at[srcv.at[_SCH - 1]], rows1, sem1)
        pltpu.make_async_copy(y_hbm.at[srcv.at[_SCH - 2]], rows0, sem0).wait()
        pltpu.sync_copy(rows0, acc.at[dstv.at[_SCH - 2]], add=True)
        pltpu.make_async_copy(y_hbm.at[srcv.at[_SCH - 1]], rows1, sem1).wait()
        pltpu.sync_copy(rows1, acc.at[dstv.at[_SCH - 1]], add=True)

    plsc.subcore_barrier()

    @pl.when(sid < 15)
    def _():
        pltpu.sync_copy(acc.at[pl.ds(base, _ST)],
                        out_hbm.at[cid, pl.ds(base, _ST)])

    @pl.when(sid == 15)
    def _():
        pltpu.sync_copy(acc.at[pl.ds(9600, 400)],
                        out_hbm.at[cid, pl.ds(9600, 400)])


def _m1_body(degp_ref, x_ref, w_ref, y_ref, dinv_ref):
    deg = degp_ref[0, 0, :] + degp_ref[0, 1, :] + 1.0
    dinv = lax.rsqrt(deg)
    xw = jnp.dot(x_ref[...], w_ref[...], preferred_element_type=jnp.float32)
    y_ref[...] = xw * dinv[:, None]
    dinv_ref[0, 0, :] = dinv


_m1 = pl.pallas_call(
    _m1_body,
    grid=(_NB,),
    in_specs=[
        pl.BlockSpec((1, 2, _BN), lambda i: (i, 0, 0)),
        pl.BlockSpec((_BN, _D), lambda i: (i, 0)),
        pl.BlockSpec((_D, _D), lambda i: (0, 0)),
    ],
    out_specs=[
        pl.BlockSpec((_BN, _D), lambda i: (i, 0)),
        pl.BlockSpec((1, 1, _BN), lambda i: (i, 0, 0)),
    ],
    out_shape=[
        jax.ShapeDtypeStruct((_N, _D), jnp.float32),
        jax.ShapeDtypeStruct((_NB, 1, _BN), jnp.float32),
    ],
)


def _m2_body(p_ref, y1_ref, dinv_ref, b1_ref, w_ref, y2_ref):
    dinv = dinv_ref[0, 0, :]
    h = (p_ref[0] + p_ref[1] + y1_ref[...]) * dinv[:, None] + b1_ref[...]
    h = jnp.maximum(h, 0.0)
    xw = jnp.dot(h, w_ref[...], preferred_element_type=jnp.float32)
    y2_ref[...] = xw * dinv[:, None]


_m2 = pl.pallas_call(
    _m2_body,
    grid=(_NB,),
    in_specs=[
        pl.BlockSpec((2, _BN, _D), lambda i: (0, i, 0)),
        pl.BlockSpec((_BN, _D), lambda i: (i, 0)),
        pl.BlockSpec((1, 1, _BN), lambda i: (i, 0, 0)),
        pl.BlockSpec((1, _D), lambda i: (0, 0)),
        pl.BlockSpec((_D, _D), lambda i: (0, 0)),
    ],
    out_specs=pl.BlockSpec((_BN, _D), lambda i: (i, 0)),
    out_shape=jax.ShapeDtypeStruct((_N, _D), jnp.float32),
)


def _m3_body(p_ref, y2_ref, dinv_ref, b2_ref, batch_ref, wc_ref, bc_ref,
             out_ref, psum, cnt):
    i = pl.program_id(0)

    @pl.when(i == 0)
    def _():
        psum[...] = jnp.zeros_like(psum)
        cnt[...] = jnp.zeros_like(cnt)

    dinv = dinv_ref[0, 0, :]
    h = (p_ref[0] + p_ref[1] + y2_ref[...]) * dinv[:, None] + b2_ref[...]
    h = jnp.maximum(h, 0.0)
    bb = batch_ref[0, 0, :]
    sel = (lax.broadcasted_iota(jnp.int32, (_G, _BN), 0) == bb[None, :])
    sel = sel.astype(jnp.float32)
    psum[...] += jnp.dot(sel, h, preferred_element_type=jnp.float32)
    cnt[...] += jnp.broadcast_to(jnp.sum(sel, axis=1, keepdims=True), (_G, _D))

    @pl.when(i == _NB - 1)
    def _():
        pooled = psum[...] / jnp.maximum(cnt[...], 1.0)
        out_ref[...] = lax.dot_general(
            pooled, wc_ref[...], (((1,), (1,)), ((), ())),
            preferred_element_type=jnp.float32) + bc_ref[...]


_m3 = pl.pallas_call(
    _m3_body,
    grid=(_NB,),
    in_specs=[
        pl.BlockSpec((2, _BN, _D), lambda i: (0, i, 0)),
        pl.BlockSpec((_BN, _D), lambda i: (i, 0)),
        pl.BlockSpec((1, 1, _BN), lambda i: (i, 0, 0)),
        pl.BlockSpec((1, _D), lambda i: (0, 0)),
        pl.BlockSpec((1, 1, _BN), lambda i: (i, 0, 0)),
        pl.BlockSpec((_C, _D), lambda i: (0, 0)),
        pl.BlockSpec((1, _C), lambda i: (0, 0)),
    ],
    out_specs=pl.BlockSpec((_G, _C), lambda i: (0, 0)),
    out_shape=jax.ShapeDtypeStruct((_G, _C), jnp.float32),
    scratch_shapes=[
        pltpu.VMEM((_G, _D), jnp.float32),
        pltpu.VMEM((_G, _D), jnp.float32),
    ],
)


def kernel(x, edge_index, batch, W1, b1, W2, b2, Wc, bc):
    src = edge_index[0].reshape(_NW, _NCH, _KC)
    dst = edge_index[1].reshape(_NW, _NCH, _KC)
    degp = _deg_kernel(dst)
    degp_t = degp[:, :_N].reshape(2, _NB, _BN).transpose(1, 0, 2)
    y1, dinv3 = _m1(degp_t, x, W1)
    p1 = _agg_kernel(y1, src, dst)
    y2 = _m2(p1, y1, dinv3, b1.reshape(1, _D), W2)
    p2 = _agg_kernel(y2, src, dst)
    batch3 = batch.reshape(_NB, 1, _BN)
    return _m3(p2, y2, dinv3, b2.reshape(1, _D), batch3, Wc,
               bc.reshape(1, _C))


# speedup vs baseline: 31.1005x; 1.0340x over previous
import functools

import jax
import jax.numpy as jnp
from jax import lax
from jax.experimental import pallas as pl
from jax.experimental.pallas import tpu as pltpu
from jax.experimental.pallas import tpu_sc as plsc

_N = 10000
_E = 320000
_D = 128
_G = 64
_C = 32

_NB = 25
_BN = _N // _NB
_NW = 32
_EW = _E // _NW
_KC = 80
_NCH = _EW // _KC
_NP = 10240
_ST = 640

_sc_mesh = plsc.VectorSubcoreMesh(core_axis_name="c", subcore_axis_name="s")


@functools.partial(
    pl.kernel,
    mesh=_sc_mesh,
    out_type=jax.ShapeDtypeStruct((2, _NP), jnp.float32),
    scratch_types=[
        pltpu.VMEM((5, 25, _KC), jnp.int32),
        pltpu.VMEM((_KC,), jnp.float32),
        pltpu.VMEM((_ST,), jnp.float32),
        pltpu.VMEM_SHARED((_NP,), jnp.float32)
    ],
)
def _deg_kernel(dst_hbm, out_hbm, dstv, onesv, zb, hist):
    cid = lax.axis_index("c")
    sid = lax.axis_index("s")
    wid = cid * 16 + sid
    pltpu.sync_copy(dst_hbm.at[wid], dstv)

    def _fill_ones(i, carry):
        onesv[pl.ds(i * 16, 16)] = jnp.full((16,), 1.0, jnp.float32)
        return carry

    lax.fori_loop(0, _KC // 16, _fill_ones, 0)

    def _fill_zero(i, carry):
        zb[pl.ds(i * 16, 16)] = jnp.zeros((16,), jnp.float32)
        return carry

    lax.fori_loop(0, _ST // 16, _fill_zero, 0)

    pltpu.sync_copy(zb, hist.at[pl.ds(sid * _ST, _ST)])
    plsc.subcore_barrier()

    def _sstep(ss, carry):
        def _step(j, carry2):
            pltpu.sync_copy(onesv, hist.at[dstv.at[ss, j]], add=True)
            return carry2
        lax.fori_loop(0, 25, _step, 0)
        return carry

    lax.fori_loop(0, 5, _sstep, 0)
    plsc.subcore_barrier()
    pltpu.sync_copy(hist.at[pl.ds(sid * _ST, _ST)],
                    out_hbm.at[cid, pl.ds(sid * _ST, _ST)])


_NSL = 5
_SCH = _NCH // _NSL


@functools.partial(
    pl.kernel,
    mesh=_sc_mesh,
    out_type=jax.ShapeDtypeStruct((2, _N, _D), jnp.float32),
    scratch_types=[
        pltpu.VMEM((_SCH, _KC), jnp.int32),
        pltpu.VMEM((_SCH, _KC), jnp.int32),
        pltpu.VMEM((_KC, _D), jnp.float32),
        pltpu.VMEM((_KC, _D), jnp.float32),
        pltpu.VMEM((_KC, _D), jnp.float32),
        pltpu.VMEM_SHARED((_N, _D), jnp.float32),
        pltpu.SemaphoreType.DMA,
        pltpu.SemaphoreType.DMA,
        pltpu.SemaphoreType.DMA,
        pltpu.SemaphoreType.DMA,
        pltpu.SemaphoreType.DMA,
        pltpu.SemaphoreType.DMA,
    ],
)
def _agg_kernel(y_hbm, src_hbm, dst_hbm, out_hbm,
                srcv, dstv, rows0, rows1, rows2, acc,
                sg0, sg1, sg2, ss0, ss1, ss2):
    cid = lax.axis_index("c")
    sid = lax.axis_index("s")
    wid = cid * 16 + sid
    rows = (rows0, rows1, rows2)
    sg = (sg0, sg1, sg2)
    ss = (ss0, ss1, ss2)

    def _gath(k, b):
        pltpu.async_copy(y_hbm.at[srcv.at[k]], rows[b], sg[b])

    def _gwait(k, b):
        pltpu.make_async_copy(y_hbm.at[srcv.at[k]], rows[b], sg[b]).wait()

    def _scat(k, b):
        pltpu.async_copy(rows[b], acc.at[dstv.at[k]], ss[b], add=True)

    def _swait(k, b):
        pltpu.make_async_copy(rows[b], acc.at[dstv.at[k]], ss[b]).wait()

    def _z(i, carry):
        for j in range(_D // 16):
            rows0[i, pl.ds(j * 16, 16)] = jnp.zeros((16,), jnp.float32)
        return carry

    lax.fori_loop(0, _KC, _z, 0)
    base = sid * _ST

    @pl.when(sid < 15)
    def _():
        for q in range(8):
            pltpu.sync_copy(rows0, acc.at[pl.ds(base + q * _KC, _KC)])

    @pl.when(sid == 15)
    def _():
        for q in range(5):
            pltpu.sync_copy(rows0, acc.at[pl.ds(9600 + q * _KC, _KC)])

    plsc.subcore_barrier()

    for s in range(_NSL):
        pltpu.sync_copy(src_hbm.at[wid, s], srcv)
        pltpu.sync_copy(dst_hbm.at[wid, s], dstv)
        _gath(0, 0)
        _gath(1, 1)
        _gwait(0, 0)
        _scat(0, 0)
        _gath(2, 2)
        _gwait(1, 1)
        _scat(1, 1)
        _swait(0, 0)
        _gath(3, 0)

        def _tri(t, carry):
            k = 3 * t + 2
            _gwait(k, 2)
            _scat(k, 2)
            _swait(k - 1, 1)
            _gath(k + 2, 1)
            _gwait(k + 1, 0)
            _scat(k + 1, 0)
            _swait(k, 2)
            _gath(k + 3, 2)
            _gwait(k + 2, 1)
            _scat(k + 2, 1)
            _swait(k + 1, 0)
            _gath(k + 4, 0)
            return carry

        lax.fori_loop(0, (_SCH - 4) // 3, _tri, 0)
        _gwait(_SCH - 2, 2)
        _scat(_SCH - 2, 2)
        _gwait(_SCH - 1, 0)
        _scat(_SCH - 1, 0)
        _swait(_SCH - 3, 1)
        _swait(_SCH - 2, 2)
        _swait(_SCH - 1, 0)

    plsc.subcore_barrier()

    @pl.when(sid < 15)
    def _():
        pltpu.sync_copy(acc.at[pl.ds(base, _ST)],
                        out_hbm.at[cid, pl.ds(base, _ST)])

    @pl.when(sid == 15)
    def _():
        pltpu.sync_copy(acc.at[pl.ds(9600, 400)],
                        out_hbm.at[cid, pl.ds(9600, 400)])


def _m1_body(degp_ref, x_ref, w_ref, y_ref, dinv_ref):
    deg = degp_ref[0, 0, :] + degp_ref[0, 1, :] + 1.0
    dinv = lax.rsqrt(deg)
    xw = jnp.dot(x_ref[...], w_ref[...], preferred_element_type=jnp.float32)
    y_ref[...] = xw * dinv[:, None]
    dinv_ref[0, 0, :] = dinv


_m1 = pl.pallas_call(
    _m1_body,
    grid=(_NB,),
    in_specs=[
        pl.BlockSpec((1, 2, _BN), lambda i: (i, 0, 0)),
        pl.BlockSpec((_BN, _D), lambda i: (i, 0)),
        pl.BlockSpec((_D, _D), lambda i: (0, 0)),
    ],
    out_specs=[
        pl.BlockSpec((_BN, _D), lambda i: (i, 0)),
        pl.BlockSpec((1, 1, _BN), lambda i: (i, 0, 0)),
    ],
    out_shape=[
        jax.ShapeDtypeStruct((_N, _D), jnp.float32),
        jax.ShapeDtypeStruct((_NB, 1, _BN), jnp.float32),
    ],
)


def _m2_body(p_ref, y1_ref, dinv_ref, b1_ref, w_ref, y2_ref):
    dinv = dinv_ref[0, 0, :]
    h = (p_ref[0] + p_ref[1] + y1_ref[...]) * dinv[:, None] + b1_ref[...]
    h = jnp.maximum(h, 0.0)
    xw = jnp.dot(h, w_ref[...], preferred_element_type=jnp.float32)
    y2_ref[...] = xw * dinv[:, None]


_m2 = pl.pallas_call(
    _m2_body,
    grid=(_NB,),
    in_specs=[
        pl.BlockSpec((2, _BN, _D), lambda i: (0, i, 0)),
        pl.BlockSpec((_BN, _D), lambda i: (i, 0)),
        pl.BlockSpec((1, 1, _BN), lambda i: (i, 0, 0)),
        pl.BlockSpec((1, _D), lambda i: (0, 0)),
        pl.BlockSpec((_D, _D), lambda i: (0, 0)),
    ],
    out_specs=pl.BlockSpec((_BN, _D), lambda i: (i, 0)),
    out_shape=jax.ShapeDtypeStruct((_N, _D), jnp.float32),
)


def _m3_body(p_ref, y2_ref, dinv_ref, b2_ref, batch_ref, wc_ref, bc_ref,
             out_ref, psum, cnt):
    i = pl.program_id(0)

    @pl.when(i == 0)
    def _():
        psum[...] = jnp.zeros_like(psum)
        cnt[...] = jnp.zeros_like(cnt)

    dinv = dinv_ref[0, 0, :]
    h = (p_ref[0] + p_ref[1] + y2_ref[...]) * dinv[:, None] + b2_ref[...]
    h = jnp.maximum(h, 0.0)
    bb = batch_ref[0, 0, :]
    sel = (lax.broadcasted_iota(jnp.int32, (_G, _BN), 0) == bb[None, :])
    sel = sel.astype(jnp.float32)
    psum[...] += jnp.dot(sel, h, preferred_element_type=jnp.float32)
    cnt[...] += jnp.broadcast_to(jnp.sum(sel, axis=1, keepdims=True), (_G, _D))

    @pl.when(i == _NB - 1)
    def _():
        pooled = psum[...] / jnp.maximum(cnt[...], 1.0)
        out_ref[...] = lax.dot_general(
            pooled, wc_ref[...], (((1,), (1,)), ((), ())),
            preferred_element_type=jnp.float32) + bc_ref[...]


_m3 = pl.pallas_call(
    _m3_body,
    grid=(_NB,),
    in_specs=[
        pl.BlockSpec((2, _BN, _D), lambda i: (0, i, 0)),
        pl.BlockSpec((_BN, _D), lambda i: (i, 0)),
        pl.BlockSpec((1, 1, _BN), lambda i: (i, 0, 0)),
        pl.BlockSpec((1, _D), lambda i: (0, 0)),
        pl.BlockSpec((1, 1, _BN), lambda i: (i, 0, 0)),
        pl.BlockSpec((_C, _D), lambda i: (0, 0)),
        pl.BlockSpec((1, _C), lambda i: (0, 0)),
    ],
    out_specs=pl.BlockSpec((_G, _C), lambda i: (0, 0)),
    out_shape=jax.ShapeDtypeStruct((_G, _C), jnp.float32),
    scratch_shapes=[
        pltpu.VMEM((_G, _D), jnp.float32),
        pltpu.VMEM((_G, _D), jnp.float32),
    ],
)


def kernel(x, edge_index, batch, W1, b1, W2, b2, Wc, bc):
    src = edge_index[0].reshape(_NW, _NSL, _SCH, _KC)
    dst = edge_index[1].reshape(_NW, _NSL, _SCH, _KC)
    degp = _deg_kernel(dst)
    degp_t = degp[:, :_N].reshape(2, _NB, _BN).transpose(1, 0, 2)
    y1, dinv3 = _m1(degp_t, x, W1)
    p1 = _agg_kernel(y1, src, dst)
    y2 = _m2(p1, y1, dinv3, b1.reshape(1, _D), W2)
    p2 = _agg_kernel(y2, src, dst)
    batch3 = batch.reshape(_NB, 1, _BN)
    return _m3(p2, y2, dinv3, b2.reshape(1, _D), batch3, Wc,
               bc.reshape(1, _C))
